# baseline (speedup 1.0000x reference)
import functools

import jax
import jax.numpy as jnp
from jax import lax
from jax.experimental import pallas as pl
from jax.experimental.pallas import tpu as pltpu
from jax.experimental.pallas import tpu_sc as plsc

_N = 6_400_000
_T = 4
_G = 512
_NC = 2
_NS = 16
_NW = _NC * _NS
_TILE = 128
_NTILES = _N // _TILE
_TPB = 40
_BR = _TPB * _TILE
_BRP = _BR + 16
_BE = _BR * _T
_NBLK = _NTILES // _TPB
_BLK_Q, _BLK_R = divmod(_NBLK, _NW)
_GT = _G * 16
_GPW = _G // _NW
_BS_ITERS = 13

_mesh = plsc.VectorSubcoreMesh(core_axis_name="c", subcore_axis_name="s")
_params = pltpu.CompilerParams(needs_layout_passes=False)


def _wid():
    return lax.axis_index("s") * _NC + lax.axis_index("c")


def _nblk(wid):
    return _BLK_Q + (wid < _BLK_R).astype(jnp.int32)


def _sload(ref, i):
    return ref[pl.ds(i, 16)][0]


def _lower_bound(bbuf, bb, x, lo0):
    def body(_, c):
        lo, hi = c
        mid = (lo + hi) >> 1
        act = lo < hi
        lt = _sload(bbuf, bb + mid) < x
        lo = jnp.where(act & lt, mid + 1, lo)
        hi = jnp.where(act & (~lt), mid, hi)
        return lo, hi

    lo, _ = lax.fori_loop(0, _BS_ITERS, body, (lo0, jnp.int32(_BR)))
    return lo


def _run_bounds(p, q):
    t0 = p >> 7
    t1 = jnp.maximum(q - 1, p) >> 7
    hi1 = jnp.minimum(q, (t0 + 1) * _TILE)
    lo2 = jnp.where(t1 > t0, t1 * _TILE, q)
    return t0, t1, hi1, lo2


def _issue_in(pred_hbm, batch_hbm, pbuf, bbuf, sem, wid, k):
    blk = wid + k * _NW
    off = k & 1
    rbase = blk * _BR
    ebase = rbase * _T
    pltpu.async_copy(pred_hbm.at[pl.ds(ebase, _BE)],
                     pbuf.at[pl.ds(off * _BE, _BE)], sem)
    pltpu.async_copy(batch_hbm.at[pl.ds(rbase, _BR)],
                     bbuf.at[pl.ds(off * _BRP, _BR)], sem)


def _wait_in(pred_hbm, batch_hbm, pbuf, bbuf, sem):
    pltpu.make_async_copy(pred_hbm.at[pl.ds(0, _BE)],
                          pbuf.at[pl.ds(0, _BE)], sem).wait()
    pltpu.make_async_copy(batch_hbm.at[pl.ds(0, _BR)],
                          bbuf.at[pl.ds(0, _BR)], sem).wait()


@functools.partial(
    pl.kernel,
    out_type=jax.ShapeDtypeStruct((_NW * _GT,), jnp.float32),
    mesh=_mesh,
    compiler_params=_params,
    scratch_types=[
        pltpu.VMEM((2 * _BE,), jnp.float32),
        pltpu.VMEM((2 * _BRP,), jnp.int32),
        pltpu.VMEM((_GT,), jnp.float32),
        pltpu.SemaphoreType.DMA,
    ],
)
def _partial_sums(pred_hbm, batch_hbm, out_hbm, pbuf, bbuf, table, sem):
    wid = _wid()
    iota = lax.iota(jnp.int32, 16)
    zeros16 = jnp.zeros((16,), jnp.float32)
    nblk = _nblk(wid)

    def zero_body(g, carry):
        table[pl.ds(g * 16, 16)] = zeros16
        return carry

    lax.fori_loop(0, _G, zero_body, 0)

    def compute(k):
        off = k & 1
        pb = off * _BE
        bb = off * _BRP

        def masked_tile(tt, lo, hi, accs):
            base = pb + tt * 512
            out = list(accs)
            for v in range(8):
                rows = tt * _TILE + v * 16 + iota
                m = (rows >= lo) & (rows < hi)
                for j in range(_T):
                    out[j] = out[j] + jnp.where(
                        m, pbuf[pl.ds(base + j * _TILE + v * 16, 16)], 0.0)
            return tuple(out)

        def full_tile(tt, accs):
            base = pb + tt * 512
            out = list(accs)
            for j in range(_T):
                for v in range(8):
                    out[j] = out[j] + pbuf[
                        pl.ds(base + j * _TILE + v * 16, 16)]
            return tuple(out)

        g0 = _sload(bbuf, bb)
        g1 = _sload(bbuf, bb + _BR - 1)

        def run_body(r, p):
            g = g0 + r
            q = _lower_bound(bbuf, bb, g + 1, p)
            t0, t1, hi1, lo2 = _run_bounds(p, q)
            accs = (zeros16, zeros16, zeros16, zeros16)
            accs = masked_tile(t0, p, hi1, accs)
            accs = lax.fori_loop(t0 + 1, t1, full_tile, accs)
            accs = masked_tile(t1, lo2, q, accs)
            s0, s1, s2, s3 = (jnp.sum(a) for a in accs)
            cnt = (q - p).astype(jnp.float32)
            upd = jnp.where(
                iota == 0, s0,
                jnp.where(iota == 1, s1,
                          jnp.where(iota == 2, s2,
                                    jnp.where(iota == 3, s3,
                                              jnp.where(iota == 4, cnt,
                                                        0.0)))))
            table[pl.ds(g * 16, 16)] = table[pl.ds(g * 16, 16)] + upd
            return q

        lax.fori_loop(0, g1 - g0 + 1, run_body, jnp.int32(0))

    _issue_in(pred_hbm, batch_hbm, pbuf, bbuf, sem, wid, jnp.int32(0))
    _issue_in(pred_hbm, batch_hbm, pbuf, bbuf, sem, wid, jnp.int32(1))

    def block_body(k, carry):
        _wait_in(pred_hbm, batch_hbm, pbuf, bbuf, sem)
        compute(k)
        _issue_in(pred_hbm, batch_hbm, pbuf, bbuf, sem, wid,
                  jnp.minimum(k + 2, nblk - 1))
        return carry

    lax.fori_loop(0, nblk, block_body, 0)
    _wait_in(pred_hbm, batch_hbm, pbuf, bbuf, sem)
    _wait_in(pred_hbm, batch_hbm, pbuf, bbuf, sem)
    pltpu.sync_copy(table, out_hbm.at[pl.ds(wid * _GT, _GT)])


@functools.partial(
    pl.kernel,
    out_type=jax.ShapeDtypeStruct((_GT,), jnp.float32),
    mesh=_mesh,
    compiler_params=_params,
    scratch_types=[
        pltpu.VMEM((_NW * _GPW * 16,), jnp.float32),
        pltpu.VMEM((_GPW * 16,), jnp.float32),
        pltpu.VMEM((32,), jnp.float32),
        pltpu.VMEM((_GPW * 16,), jnp.float32),
        pltpu.SemaphoreType.DMA,
    ],
)
def _correction(part_hbm, st_hbm, ms_hbm, corr_hbm, part_v, st_v, ms_v,
                out_v, sem):
    wid = _wid()
    iota = lax.iota(jnp.int32, 16)
    i4 = iota % 4
    seg = _GPW * 16

    handles = []
    for w2 in range(_NW):
        handles.append(pltpu.async_copy(
            part_hbm.at[pl.ds(w2 * _GT + wid * seg, seg)],
            part_v.at[pl.ds(w2 * seg, seg)], sem))
    handles.append(pltpu.async_copy(st_hbm.at[pl.ds(wid * seg, seg)], st_v,
                                    sem))
    handles.append(pltpu.async_copy(ms_hbm, ms_v, sem))
    for h in handles:
        h.wait()

    def seg_body(j, carry):
        def add_w(w2, acc):
            return acc + part_v[pl.ds(w2 * seg + j * 16, 16)]

        acc = lax.fori_loop(0, _NW, add_w, jnp.zeros((16,), jnp.float32))
        s0 = jnp.sum(jnp.where(iota == 0, acc, 0.0))
        s1 = jnp.sum(jnp.where(iota == 1, acc, 0.0))
        s2 = jnp.sum(jnp.where(iota == 2, acc, 0.0))
        s3 = jnp.sum(jnp.where(iota == 3, acc, 0.0))
        cnt = jnp.sum(jnp.where(iota == 4, acc, 0.0))
        psum = jnp.where(i4 == 0, s0,
                         jnp.where(i4 == 1, s1,
                                   jnp.where(i4 == 2, s2, s3)))
        st = st_v[pl.ds(j * 16, 16)]
        meanv = ms_v[pl.ds(0, 16)]
        stdv = ms_v[pl.ds(16, 16)]
        corr = ((st - cnt * meanv) / stdv - psum) / cnt
        out_v[pl.ds(j * 16, 16)] = corr
        return carry

    lax.fori_loop(0, _GPW, seg_body, 0)
    pltpu.sync_copy(out_v, corr_hbm.at[pl.ds(wid * seg, seg)])


@functools.partial(
    pl.kernel,
    out_type=jax.ShapeDtypeStruct((_N * _T,), jnp.float32),
    mesh=_mesh,
    compiler_params=_params,
    scratch_types=[
        pltpu.VMEM((2 * _BE,), jnp.float32),
        pltpu.VMEM((2 * _BRP,), jnp.int32),
        pltpu.VMEM((2 * _BE,), jnp.float32),
        pltpu.VMEM((_GT,), jnp.float32),
        pltpu.SemaphoreType.DMA,
        pltpu.SemaphoreType.DMA,
    ],
)
def _apply(pred_hbm, batch_hbm, corr_hbm, out_hbm, pbuf, bbuf, obuf, corr_v,
           sem_in, sem_out):
    wid = _wid()
    iota = lax.iota(jnp.int32, 16)
    nblk = _nblk(wid)
    pltpu.sync_copy(corr_hbm, corr_v)

    def compute_and_flush(k):
        off = k & 1
        pb = off * _BE
        bb = off * _BRP

        def masked_tile(tt, lo, hi, cj, seed):
            for v in range(8):
                rows = tt * _TILE + v * 16 + iota
                m = (rows >= lo) & (rows < hi)
                for j in range(_T):
                    o = pb + tt * 512 + j * _TILE + v * 16
                    src = jnp.where(seed, pbuf[pl.ds(o, 16)],
                                    obuf[pl.ds(o, 16)])
                    obuf[pl.ds(o, 16)] = src + jnp.where(m, cj[j], 0.0)

        g0 = _sload(bbuf, bb)
        g1 = _sload(bbuf, bb + _BR - 1)

        def run_body(r, p):
            g = g0 + r
            q = _lower_bound(bbuf, bb, g + 1, p)
            t0, t1, hi1, lo2 = _run_bounds(p, q)
            cvec = corr_v[pl.ds(g * 16, 16)]
            cj = tuple(jnp.full((16,), cvec[j], jnp.float32)
                       for j in range(_T))

            def full_tile(tt, carry2):
                for j in range(_T):
                    for v in range(8):
                        o = pb + tt * 512 + j * _TILE + v * 16
                        obuf[pl.ds(o, 16)] = pbuf[pl.ds(o, 16)] + cj[j]
                return carry2

            masked_tile(t0, p, hi1, cj, p == t0 * _TILE)
            lax.fori_loop(t0 + 1, t1, full_tile, 0)
            masked_tile(t1, lo2, q, cj, t1 > t0)
            return q

        lax.fori_loop(0, g1 - g0 + 1, run_body, jnp.int32(0))

        rbase = (wid + k * _NW) * _BR
        ebase = rbase * _T
        pltpu.async_copy(obuf.at[pl.ds(pb, _BE)],
                         out_hbm.at[pl.ds(ebase, _BE)], sem_out)

    def wait_out():
        pltpu.make_async_copy(obuf.at[pl.ds(0, _BE)],
                              out_hbm.at[pl.ds(0, _BE)], sem_out).wait()

    _issue_in(pred_hbm, batch_hbm, pbuf, bbuf, sem_in, wid, jnp.int32(0))
    _issue_in(pred_hbm, batch_hbm, pbuf, bbuf, sem_in, wid, jnp.int32(1))

    for k0 in (0, 1):
        _wait_in(pred_hbm, batch_hbm, pbuf, bbuf, sem_in)
        compute_and_flush(jnp.int32(k0))
        _issue_in(pred_hbm, batch_hbm, pbuf, bbuf, sem_in, wid,
                  jnp.minimum(jnp.int32(k0 + 2), nblk - 1))

    def block_body(k, carry):
        _wait_in(pred_hbm, batch_hbm, pbuf, bbuf, sem_in)
        wait_out()
        compute_and_flush(k)
        _issue_in(pred_hbm, batch_hbm, pbuf, bbuf, sem_in, wid,
                  jnp.minimum(k + 2, nblk - 1))
        return carry

    lax.fori_loop(2, nblk, block_body, 0)
    wait_out()
    wait_out()
    _wait_in(pred_hbm, batch_hbm, pbuf, bbuf, sem_in)
    _wait_in(pred_hbm, batch_hbm, pbuf, bbuf, sem_in)


def kernel(pred, batch, sum_target, mean, std):
    pred_flat = pred.reshape(_NTILES, _TILE, _T).transpose(0, 2, 1).reshape(-1)
    st16 = jnp.tile(sum_target, (1, 4)).reshape(-1)
    ms = jnp.concatenate([jnp.tile(mean, 4), jnp.tile(std, 4)])
    part = _partial_sums(pred_flat, batch)
    corr = _correction(part, st16, ms)
    out = _apply(pred_flat, batch, corr)
    return out.reshape(_NTILES, _T, _TILE).transpose(0, 2, 1).reshape(_N, _T)


# speedup vs baseline: 1.0085x; 1.0085x over previous
import functools

import jax
import jax.numpy as jnp
from jax import lax
from jax.experimental import pallas as pl
from jax.experimental.pallas import tpu as pltpu
from jax.experimental.pallas import tpu_sc as plsc

_N = 6_400_000
_T = 4
_G = 512
_NC = 2
_NS = 16
_NW = _NC * _NS
_TILE = 128
_NTILES = _N // _TILE
_TPB = 50
_BR = _TPB * _TILE
_BRP = _BR + 16
_BE = _BR * _T
_NBLK = _NTILES // _TPB
_BLK_Q, _BLK_R = divmod(_NBLK, _NW)
_GT = _G * 16
_GPW = _G // _NW
_BS_ITERS = 13

_mesh = plsc.VectorSubcoreMesh(core_axis_name="c", subcore_axis_name="s")
_params = pltpu.CompilerParams(needs_layout_passes=False)


def _wid():
    return lax.axis_index("s") * _NC + lax.axis_index("c")


def _nblk(wid):
    return _BLK_Q + (wid < _BLK_R).astype(jnp.int32)


def _sload(ref, i):
    return ref[pl.ds(i, 16)][0]


def _lower_bound(bbuf, bb, x, lo0):
    def body(_, c):
        lo, hi = c
        mid = (lo + hi) >> 1
        act = lo < hi
        lt = _sload(bbuf, bb + mid) < x
        lo = jnp.where(act & lt, mid + 1, lo)
        hi = jnp.where(act & (~lt), mid, hi)
        return lo, hi

    lo, _ = lax.fori_loop(0, _BS_ITERS, body, (lo0, jnp.int32(_BR)))
    return lo


def _run_bounds(p, q):
    t0 = p >> 7
    t1 = jnp.maximum(q - 1, p) >> 7
    hi1 = jnp.minimum(q, (t0 + 1) * _TILE)
    lo2 = jnp.where(t1 > t0, t1 * _TILE, q)
    return t0, t1, hi1, lo2


def _issue_in(pred_hbm, batch_hbm, pbuf, bbuf, sem, wid, k):
    blk = wid + k * _NW
    off = k & 1
    rbase = blk * _BR
    ebase = rbase * _T
    pltpu.async_copy(pred_hbm.at[pl.ds(ebase, _BE)],
                     pbuf.at[pl.ds(off * _BE, _BE)], sem)
    pltpu.async_copy(batch_hbm.at[pl.ds(rbase, _BR)],
                     bbuf.at[pl.ds(off * _BRP, _BR)], sem)


def _wait_in(pred_hbm, batch_hbm, pbuf, bbuf, sem):
    pltpu.make_async_copy(pred_hbm.at[pl.ds(0, _BE)],
                          pbuf.at[pl.ds(0, _BE)], sem).wait()
    pltpu.make_async_copy(batch_hbm.at[pl.ds(0, _BR)],
                          bbuf.at[pl.ds(0, _BR)], sem).wait()


@functools.partial(
    pl.kernel,
    out_type=jax.ShapeDtypeStruct((_NW * _GT,), jnp.float32),
    mesh=_mesh,
    compiler_params=_params,
    scratch_types=[
        pltpu.VMEM((2 * _BE,), jnp.float32),
        pltpu.VMEM((2 * _BRP,), jnp.int32),
        pltpu.VMEM((_GT,), jnp.float32),
        pltpu.SemaphoreType.DMA,
    ],
)
def _partial_sums(pred_hbm, batch_hbm, out_hbm, pbuf, bbuf, table, sem):
    wid = _wid()
    iota = lax.iota(jnp.int32, 16)
    zeros16 = jnp.zeros((16,), jnp.float32)
    nblk = _nblk(wid)

    def zero_body(g, carry):
        table[pl.ds(g * 16, 16)] = zeros16
        return carry

    lax.fori_loop(0, _G, zero_body, 0)

    def compute(k):
        off = k & 1
        pb = off * _BE
        bb = off * _BRP

        def masked_tile(tt, lo, hi, accs):
            base = pb + tt * 512
            out = list(accs)
            for v in range(8):
                rows = tt * _TILE + v * 16 + iota
                m = (rows >= lo) & (rows < hi)
                for j in range(_T):
                    out[j] = out[j] + jnp.where(
                        m, pbuf[pl.ds(base + j * _TILE + v * 16, 16)], 0.0)
            return tuple(out)

        def full_tile(tt, accs):
            base = pb + tt * 512
            out = list(accs)
            for j in range(_T):
                for v in range(8):
                    out[j] = out[j] + pbuf[
                        pl.ds(base + j * _TILE + v * 16, 16)]
            return tuple(out)

        g0 = _sload(bbuf, bb)
        g1 = _sload(bbuf, bb + _BR - 1)

        def run_body(r, p):
            g = g0 + r
            q = _lower_bound(bbuf, bb, g + 1, p)
            t0, t1, hi1, lo2 = _run_bounds(p, q)
            accs = (zeros16, zeros16, zeros16, zeros16)
            accs = masked_tile(t0, p, hi1, accs)
            accs = lax.fori_loop(t0 + 1, t1, full_tile, accs)
            accs = masked_tile(t1, lo2, q, accs)
            s0, s1, s2, s3 = (jnp.sum(a) for a in accs)
            cnt = (q - p).astype(jnp.float32)
            upd = jnp.where(
                iota == 0, s0,
                jnp.where(iota == 1, s1,
                          jnp.where(iota == 2, s2,
                                    jnp.where(iota == 3, s3,
                                              jnp.where(iota == 4, cnt,
                                                        0.0)))))
            table[pl.ds(g * 16, 16)] = table[pl.ds(g * 16, 16)] + upd
            return q

        lax.fori_loop(0, g1 - g0 + 1, run_body, jnp.int32(0))

    _issue_in(pred_hbm, batch_hbm, pbuf, bbuf, sem, wid, jnp.int32(0))
    _issue_in(pred_hbm, batch_hbm, pbuf, bbuf, sem, wid, jnp.int32(1))

    def block_body(k, carry):
        _wait_in(pred_hbm, batch_hbm, pbuf, bbuf, sem)
        compute(k)
        _issue_in(pred_hbm, batch_hbm, pbuf, bbuf, sem, wid,
                  jnp.minimum(k + 2, nblk - 1))
        return carry

    lax.fori_loop(0, nblk, block_body, 0)
    _wait_in(pred_hbm, batch_hbm, pbuf, bbuf, sem)
    _wait_in(pred_hbm, batch_hbm, pbuf, bbuf, sem)
    pltpu.sync_copy(table, out_hbm.at[pl.ds(wid * _GT, _GT)])


@functools.partial(
    pl.kernel,
    out_type=jax.ShapeDtypeStruct((_GT,), jnp.float32),
    mesh=_mesh,
    compiler_params=_params,
    scratch_types=[
        pltpu.VMEM((_NW * _GPW * 16,), jnp.float32),
        pltpu.VMEM((_GPW * 16,), jnp.float32),
        pltpu.VMEM((32,), jnp.float32),
        pltpu.VMEM((_GPW * 16,), jnp.float32),
        pltpu.SemaphoreType.DMA,
    ],
)
def _correction(part_hbm, st_hbm, ms_hbm, corr_hbm, part_v, st_v, ms_v,
                out_v, sem):
    wid = _wid()
    iota = lax.iota(jnp.int32, 16)
    i4 = iota % 4
    seg = _GPW * 16

    handles = []
    for w2 in range(_NW):
        handles.append(pltpu.async_copy(
            part_hbm.at[pl.ds(w2 * _GT + wid * seg, seg)],
            part_v.at[pl.ds(w2 * seg, seg)], sem))
    handles.append(pltpu.async_copy(st_hbm.at[pl.ds(wid * seg, seg)], st_v,
                                    sem))
    handles.append(pltpu.async_copy(ms_hbm, ms_v, sem))
    for h in handles:
        h.wait()

    def seg_body(j, carry):
        def add_w(w2, acc):
            return acc + part_v[pl.ds(w2 * seg + j * 16, 16)]

        acc = lax.fori_loop(0, _NW, add_w, jnp.zeros((16,), jnp.float32))
        s0 = jnp.sum(jnp.where(iota == 0, acc, 0.0))
        s1 = jnp.sum(jnp.where(iota == 1, acc, 0.0))
        s2 = jnp.sum(jnp.where(iota == 2, acc, 0.0))
        s3 = jnp.sum(jnp.where(iota == 3, acc, 0.0))
        cnt = jnp.sum(jnp.where(iota == 4, acc, 0.0))
        psum = jnp.where(i4 == 0, s0,
                         jnp.where(i4 == 1, s1,
                                   jnp.where(i4 == 2, s2, s3)))
        st = st_v[pl.ds(j * 16, 16)]
        meanv = ms_v[pl.ds(0, 16)]
        stdv = ms_v[pl.ds(16, 16)]
        corr = ((st - cnt * meanv) / stdv - psum) / cnt
        out_v[pl.ds(j * 16, 16)] = corr
        return carry

    lax.fori_loop(0, _GPW, seg_body, 0)
    pltpu.sync_copy(out_v, corr_hbm.at[pl.ds(wid * seg, seg)])


@functools.partial(
    pl.kernel,
    out_type=jax.ShapeDtypeStruct((_N * _T,), jnp.float32),
    mesh=_mesh,
    compiler_params=_params,
    scratch_types=[
        pltpu.VMEM((2 * _BE,), jnp.float32),
        pltpu.VMEM((2 * _BRP,), jnp.int32),
        pltpu.VMEM((2 * _BE,), jnp.float32),
        pltpu.VMEM((_GT,), jnp.float32),
        pltpu.SemaphoreType.DMA,
        pltpu.SemaphoreType.DMA,
    ],
)
def _apply(pred_hbm, batch_hbm, corr_hbm, out_hbm, pbuf, bbuf, obuf, corr_v,
           sem_in, sem_out):
    wid = _wid()
    iota = lax.iota(jnp.int32, 16)
    nblk = _nblk(wid)
    pltpu.sync_copy(corr_hbm, corr_v)

    def compute_and_flush(k):
        off = k & 1
        pb = off * _BE
        bb = off * _BRP

        def masked_tile(tt, lo, hi, cj, seed):
            for v in range(8):
                rows = tt * _TILE + v * 16 + iota
                m = (rows >= lo) & (rows < hi)
                for j in range(_T):
                    o = pb + tt * 512 + j * _TILE + v * 16
                    src = jnp.where(seed, pbuf[pl.ds(o, 16)],
                                    obuf[pl.ds(o, 16)])
                    obuf[pl.ds(o, 16)] = src + jnp.where(m, cj[j], 0.0)

        g0 = _sload(bbuf, bb)
        g1 = _sload(bbuf, bb + _BR - 1)

        def run_body(r, p):
            g = g0 + r
            q = _lower_bound(bbuf, bb, g + 1, p)
            t0, t1, hi1, lo2 = _run_bounds(p, q)
            cvec = corr_v[pl.ds(g * 16, 16)]
            cj = tuple(jnp.full((16,), cvec[j], jnp.float32)
                       for j in range(_T))

            def full_tile(tt, carry2):
                for j in range(_T):
                    for v in range(8):
                        o = pb + tt * 512 + j * _TILE + v * 16
                        obuf[pl.ds(o, 16)] = pbuf[pl.ds(o, 16)] + cj[j]
                return carry2

            masked_tile(t0, p, hi1, cj, p == t0 * _TILE)
            lax.fori_loop(t0 + 1, t1, full_tile, 0)
            masked_tile(t1, lo2, q, cj, t1 > t0)
            return q

        lax.fori_loop(0, g1 - g0 + 1, run_body, jnp.int32(0))

        rbase = (wid + k * _NW) * _BR
        ebase = rbase * _T
        pltpu.async_copy(obuf.at[pl.ds(pb, _BE)],
                         out_hbm.at[pl.ds(ebase, _BE)], sem_out)

    def wait_out():
        pltpu.make_async_copy(obuf.at[pl.ds(0, _BE)],
                              out_hbm.at[pl.ds(0, _BE)], sem_out).wait()

    _issue_in(pred_hbm, batch_hbm, pbuf, bbuf, sem_in, wid, jnp.int32(0))
    _issue_in(pred_hbm, batch_hbm, pbuf, bbuf, sem_in, wid, jnp.int32(1))

    for k0 in (0, 1):
        _wait_in(pred_hbm, batch_hbm, pbuf, bbuf, sem_in)
        compute_and_flush(jnp.int32(k0))
        _issue_in(pred_hbm, batch_hbm, pbuf, bbuf, sem_in, wid,
                  jnp.minimum(jnp.int32(k0 + 2), nblk - 1))

    def block_body(k, carry):
        _wait_in(pred_hbm, batch_hbm, pbuf, bbuf, sem_in)
        wait_out()
        compute_and_flush(k)
        _issue_in(pred_hbm, batch_hbm, pbuf, bbuf, sem_in, wid,
                  jnp.minimum(k + 2, nblk - 1))
        return carry

    lax.fori_loop(2, nblk, block_body, 0)
    wait_out()
    wait_out()
    _wait_in(pred_hbm, batch_hbm, pbuf, bbuf, sem_in)
    _wait_in(pred_hbm, batch_hbm, pbuf, bbuf, sem_in)


def kernel(pred, batch, sum_target, mean, std):
    pred_flat = pred.reshape(_NTILES, _TILE, _T).transpose(0, 2, 1).reshape(-1)
    st16 = jnp.tile(sum_target, (1, 4)).reshape(-1)
    ms = jnp.concatenate([jnp.tile(mean, 4), jnp.tile(std, 4)])
    part = _partial_sums(pred_flat, batch)
    corr = _correction(part, st16, ms)
    out = _apply(pred_flat, batch, corr)
    return out.reshape(_NTILES, _T, _TILE).transpose(0, 2, 1).reshape(_N, _T)


# speedup vs baseline: 2.0027x; 1.9857x over previous
import functools

import jax
import jax.numpy as jnp
from jax import lax
from jax.experimental import pallas as pl
from jax.experimental.pallas import tpu as pltpu
from jax.experimental.pallas import tpu_sc as plsc

_N = 6_400_000
_T = 4
_G = 512
_NC = 2
_NS = 16
_NW = _NC * _NS
_TILE = 128
_NTILES = _N // _TILE
_TPB = 50
_BR = _TPB * _TILE
_BRP = _BR + 16
_BE = _BR * _T
_NBLK = _NTILES // _TPB
_BLK_Q, _BLK_R = divmod(_NBLK, _NW)
_GT = _G * 16
_GPW = _G // _NW
_BS_ITERS = 13

_mesh = plsc.VectorSubcoreMesh(core_axis_name="c", subcore_axis_name="s")
_params = pltpu.CompilerParams(needs_layout_passes=False)


def _wid():
    return lax.axis_index("s") * _NC + lax.axis_index("c")


def _nblk(wid):
    return _BLK_Q + (wid < _BLK_R).astype(jnp.int32)


def _sload(ref, i):
    return ref[pl.ds(i, 16)][0]


def _lower_bound(bbuf, bb, x, lo0):
    def body(_, c):
        lo, hi = c
        mid = (lo + hi) >> 1
        act = lo < hi
        lt = _sload(bbuf, bb + mid) < x
        lo = jnp.where(act & lt, mid + 1, lo)
        hi = jnp.where(act & (~lt), mid, hi)
        return lo, hi

    lo, _ = lax.fori_loop(0, _BS_ITERS, body, (lo0, jnp.int32(_BR)))
    return lo


def _run_bounds(p, q):
    t0 = p >> 7
    t1 = jnp.maximum(q - 1, p) >> 7
    hi1 = jnp.minimum(q, (t0 + 1) * _TILE)
    lo2 = jnp.where(t1 > t0, t1 * _TILE, q)
    return t0, t1, hi1, lo2


def _issue_in(pred_hbm, batch_hbm, pbuf, bbuf, sem, wid, k):
    blk = wid + k * _NW
    off = k & 1
    rbase = blk * _BR
    ebase = rbase * _T
    pltpu.async_copy(pred_hbm.at[pl.ds(ebase, _BE)],
                     pbuf.at[pl.ds(off * _BE, _BE)], sem)
    pltpu.async_copy(batch_hbm.at[pl.ds(rbase, _BR)],
                     bbuf.at[pl.ds(off * _BRP, _BR)], sem)


def _wait_in(pred_hbm, batch_hbm, pbuf, bbuf, sem):
    pltpu.make_async_copy(pred_hbm.at[pl.ds(0, _BE)],
                          pbuf.at[pl.ds(0, _BE)], sem).wait()
    pltpu.make_async_copy(batch_hbm.at[pl.ds(0, _BR)],
                          bbuf.at[pl.ds(0, _BR)], sem).wait()


@functools.partial(
    pl.kernel,
    out_type=jax.ShapeDtypeStruct((_NW * _GT,), jnp.float32),
    mesh=_mesh,
    compiler_params=_params,
    scratch_types=[
        pltpu.VMEM((2 * _BE,), jnp.float32),
        pltpu.VMEM((2 * _BRP,), jnp.int32),
        pltpu.VMEM((_GT,), jnp.float32),
        pltpu.SemaphoreType.DMA,
    ],
)
def _partial_sums(pred_hbm, batch_hbm, out_hbm, pbuf, bbuf, table, sem):
    wid = _wid()
    iota = lax.iota(jnp.int32, 16)
    zeros16 = jnp.zeros((16,), jnp.float32)
    nblk = _nblk(wid)

    def zero_body(g, carry):
        table[pl.ds(g * 16, 16)] = zeros16
        return carry

    lax.fori_loop(0, _G, zero_body, 0)

    def compute(k):
        off = k & 1
        pb = off * _BE
        bb = off * _BRP

        def masked_tile(tt, lo, hi, accs):
            base = pb + tt * 512
            out = list(accs)
            for v in range(8):
                rows = tt * _TILE + v * 16 + iota
                m = (rows >= lo) & (rows < hi)
                for j in range(_T):
                    out[j] = out[j] + jnp.where(
                        m, pbuf[pl.ds(base + j * _TILE + v * 16, 16)], 0.0)
            return tuple(out)

        def full_tile(tt, accs):
            base = pb + tt * 512
            out = list(accs)
            for j in range(_T):
                for v in range(8):
                    out[j] = out[j] + pbuf[
                        pl.ds(base + j * _TILE + v * 16, 16)]
            return tuple(out)

        g0 = _sload(bbuf, bb)
        g1 = _sload(bbuf, bb + _BR - 1)

        def run_body(r, p):
            g = g0 + r
            q = _lower_bound(bbuf, bb, g + 1, p)
            t0, t1, hi1, lo2 = _run_bounds(p, q)
            accs = (zeros16, zeros16, zeros16, zeros16)
            accs = masked_tile(t0, p, hi1, accs)
            accs = lax.fori_loop(t0 + 1, t1, full_tile, accs)
            accs = masked_tile(t1, lo2, q, accs)
            s0, s1, s2, s3 = (jnp.sum(a) for a in accs)
            cnt = (q - p).astype(jnp.float32)
            upd = jnp.where(
                iota == 0, s0,
                jnp.where(iota == 1, s1,
                          jnp.where(iota == 2, s2,
                                    jnp.where(iota == 3, s3,
                                              jnp.where(iota == 4, cnt,
                                                        0.0)))))
            table[pl.ds(g * 16, 16)] = table[pl.ds(g * 16, 16)] + upd
            return q

        lax.fori_loop(0, g1 - g0 + 1, run_body, jnp.int32(0))

    _issue_in(pred_hbm, batch_hbm, pbuf, bbuf, sem, wid, jnp.int32(0))
    _issue_in(pred_hbm, batch_hbm, pbuf, bbuf, sem, wid, jnp.int32(1))

    def block_body(k, carry):
        _wait_in(pred_hbm, batch_hbm, pbuf, bbuf, sem)
        compute(k)
        _issue_in(pred_hbm, batch_hbm, pbuf, bbuf, sem, wid,
                  jnp.minimum(k + 2, nblk - 1))
        return carry

    lax.fori_loop(0, nblk, block_body, 0)
    _wait_in(pred_hbm, batch_hbm, pbuf, bbuf, sem)
    _wait_in(pred_hbm, batch_hbm, pbuf, bbuf, sem)
    pltpu.sync_copy(table, out_hbm.at[pl.ds(wid * _GT, _GT)])


@functools.partial(
    pl.kernel,
    out_type=jax.ShapeDtypeStruct((_GT,), jnp.float32),
    mesh=_mesh,
    compiler_params=_params,
    scratch_types=[
        pltpu.VMEM((_NW * _GPW * 16,), jnp.float32),
        pltpu.VMEM((_GPW * 16,), jnp.float32),
        pltpu.VMEM((32,), jnp.float32),
        pltpu.VMEM((_GPW * 16,), jnp.float32),
        pltpu.SemaphoreType.DMA,
    ],
)
def _correction(part_hbm, st_hbm, ms_hbm, corr_hbm, part_v, st_v, ms_v,
                out_v, sem):
    wid = _wid()
    iota = lax.iota(jnp.int32, 16)
    i4 = iota % 4
    seg = _GPW * 16

    handles = []
    for w2 in range(_NW):
        handles.append(pltpu.async_copy(
            part_hbm.at[pl.ds(w2 * _GT + wid * seg, seg)],
            part_v.at[pl.ds(w2 * seg, seg)], sem))
    handles.append(pltpu.async_copy(st_hbm.at[pl.ds(wid * seg, seg)], st_v,
                                    sem))
    handles.append(pltpu.async_copy(ms_hbm, ms_v, sem))
    for h in handles:
        h.wait()

    def seg_body(j, carry):
        def add_w(w2, acc):
            return acc + part_v[pl.ds(w2 * seg + j * 16, 16)]

        acc = lax.fori_loop(0, _NW, add_w, jnp.zeros((16,), jnp.float32))
        s0 = jnp.sum(jnp.where(iota == 0, acc, 0.0))
        s1 = jnp.sum(jnp.where(iota == 1, acc, 0.0))
        s2 = jnp.sum(jnp.where(iota == 2, acc, 0.0))
        s3 = jnp.sum(jnp.where(iota == 3, acc, 0.0))
        cnt = jnp.sum(jnp.where(iota == 4, acc, 0.0))
        psum = jnp.where(i4 == 0, s0,
                         jnp.where(i4 == 1, s1,
                                   jnp.where(i4 == 2, s2, s3)))
        st = st_v[pl.ds(j * 16, 16)]
        meanv = ms_v[pl.ds(0, 16)]
        stdv = ms_v[pl.ds(16, 16)]
        corr = ((st - cnt * meanv) / stdv - psum) / cnt
        out_v[pl.ds(j * 16, 16)] = corr
        return carry

    lax.fori_loop(0, _GPW, seg_body, 0)
    pltpu.sync_copy(out_v, corr_hbm.at[pl.ds(wid * seg, seg)])


@functools.partial(
    pl.kernel,
    out_type=jax.ShapeDtypeStruct((_N * _T,), jnp.float32),
    mesh=_mesh,
    compiler_params=_params,
    scratch_types=[
        pltpu.VMEM((2 * _BE,), jnp.float32),
        pltpu.VMEM((2 * _BRP,), jnp.int32),
        pltpu.VMEM((_GT,), jnp.float32),
        pltpu.SemaphoreType.DMA,
    ],
)
def _apply(pred_hbm, batch_hbm, corr_hbm, out_hbm, pbuf, bbuf, corr_v,
           sem_in):
    wid = _wid()
    iota = lax.iota(jnp.int32, 16)
    nblk = _nblk(wid)
    pltpu.sync_copy(corr_hbm, corr_v)

    def compute(k):
        off = k & 1
        pb = off * _BE
        bb = off * _BRP

        def masked_tile(tt, lo, hi, cj):
            for v in range(8):
                rows = tt * _TILE + v * 16 + iota
                m = (rows >= lo) & (rows < hi)
                for j in range(_T):
                    o = pb + tt * 512 + j * _TILE + v * 16
                    pbuf[pl.ds(o, 16)] = (
                        pbuf[pl.ds(o, 16)] + jnp.where(m, cj[j], 0.0))

        g0 = _sload(bbuf, bb)
        g1 = _sload(bbuf, bb + _BR - 1)

        def run_body(r, p):
            g = g0 + r
            q = _lower_bound(bbuf, bb, g + 1, p)
            t0, t1, hi1, lo2 = _run_bounds(p, q)
            cvec = corr_v[pl.ds(g * 16, 16)]
            cj = tuple(jnp.full((16,), cvec[j], jnp.float32)
                       for j in range(_T))

            def full_tile(tt, carry2):
                for j in range(_T):
                    for v in range(8):
                        o = pb + tt * 512 + j * _TILE + v * 16
                        pbuf[pl.ds(o, 16)] = pbuf[pl.ds(o, 16)] + cj[j]
                return carry2

            masked_tile(t0, p, hi1, cj)
            lax.fori_loop(t0 + 1, t1, full_tile, 0)
            masked_tile(t1, lo2, q, cj)
            return q

        lax.fori_loop(0, g1 - g0 + 1, run_body, jnp.int32(0))

    _issue_in(pred_hbm, batch_hbm, pbuf, bbuf, sem_in, wid, jnp.int32(0))
    _issue_in(pred_hbm, batch_hbm, pbuf, bbuf, sem_in, wid, jnp.int32(1))

    def block_body(k, carry):
        _wait_in(pred_hbm, batch_hbm, pbuf, bbuf, sem_in)
        compute(k)
        ebase = (wid + k * _NW) * _BE
        pltpu.sync_copy(pbuf.at[pl.ds((k & 1) * _BE, _BE)],
                        out_hbm.at[pl.ds(ebase, _BE)])
        _issue_in(pred_hbm, batch_hbm, pbuf, bbuf, sem_in, wid,
                  jnp.minimum(k + 2, nblk - 1))
        return carry

    lax.fori_loop(0, nblk, block_body, 0)
    _wait_in(pred_hbm, batch_hbm, pbuf, bbuf, sem_in)
    _wait_in(pred_hbm, batch_hbm, pbuf, bbuf, sem_in)


def kernel(pred, batch, sum_target, mean, std):
    pred_flat = pred.reshape(_NTILES, _TILE, _T).transpose(0, 2, 1).reshape(-1)
    st16 = jnp.tile(sum_target, (1, 4)).reshape(-1)
    ms = jnp.concatenate([jnp.tile(mean, 4), jnp.tile(std, 4)])
    part = _partial_sums(pred_flat, batch)
    corr = _correction(part, st16, ms)
    out = _apply(pred_flat, batch, corr)
    return out.reshape(_NTILES, _T, _TILE).transpose(0, 2, 1).reshape(_N, _T)


# speedup vs baseline: 2.0109x; 1.0041x over previous
import functools

import jax
import jax.numpy as jnp
from jax import lax
from jax.experimental import pallas as pl
from jax.experimental.pallas import tpu as pltpu
from jax.experimental.pallas import tpu_sc as plsc

_N = 6_400_000
_T = 4
_G = 512
_NC = 2
_NS = 16
_NW = _NC * _NS
_TILE = 128
_NTILES = _N // _TILE
_TPB = 50
_BR = _TPB * _TILE
_BRP = _BR + 16
_BE = _BR * _T
_NBLK = _NTILES // _TPB
_BLK_Q, _BLK_R = divmod(_NBLK, _NW)
_GT = _G * 16
_GPW = _G // _NW
_BS_ITERS = 13

_mesh = plsc.VectorSubcoreMesh(core_axis_name="c", subcore_axis_name="s")
_params = pltpu.CompilerParams(needs_layout_passes=False)


def _wid():
    return lax.axis_index("s") * _NC + lax.axis_index("c")


def _nblk(wid):
    return _BLK_Q + (wid < _BLK_R).astype(jnp.int32)


def _sload(ref, i):
    return ref[pl.ds(i, 16)][0]


def _lower_bound(bbuf, bb, x, lo0):
    def body(_, c):
        lo, hi = c
        mid = (lo + hi) >> 1
        act = lo < hi
        lt = _sload(bbuf, bb + mid) < x
        lo = jnp.where(act & lt, mid + 1, lo)
        hi = jnp.where(act & (~lt), mid, hi)
        return lo, hi

    lo, _ = lax.fori_loop(0, _BS_ITERS, body, (lo0, jnp.int32(_BR)))
    return lo


def _run_bounds(p, q):
    t0 = p >> 7
    t1 = jnp.maximum(q - 1, p) >> 7
    hi1 = jnp.minimum(q, (t0 + 1) * _TILE)
    lo2 = jnp.where(t1 > t0, t1 * _TILE, q)
    return t0, t1, hi1, lo2


def _issue_in(pred_hbm, batch_hbm, pbuf, bbuf, sem, wid, k):
    blk = wid + k * _NW
    off = k & 1
    rbase = blk * _BR
    ebase = rbase * _T
    pltpu.async_copy(pred_hbm.at[pl.ds(ebase, _BE)],
                     pbuf.at[pl.ds(off * _BE, _BE)], sem)
    pltpu.async_copy(batch_hbm.at[pl.ds(rbase, _BR)],
                     bbuf.at[pl.ds(off * _BRP, _BR)], sem)


def _wait_in(pred_hbm, batch_hbm, pbuf, bbuf, sem):
    pltpu.make_async_copy(pred_hbm.at[pl.ds(0, _BE)],
                          pbuf.at[pl.ds(0, _BE)], sem).wait()
    pltpu.make_async_copy(batch_hbm.at[pl.ds(0, _BR)],
                          bbuf.at[pl.ds(0, _BR)], sem).wait()


@functools.partial(
    pl.kernel,
    out_type=jax.ShapeDtypeStruct((_NW * _GT,), jnp.float32),
    mesh=_mesh,
    compiler_params=_params,
    scratch_types=[
        pltpu.VMEM((2 * _BE,), jnp.float32),
        pltpu.VMEM((2 * _BRP,), jnp.int32),
        pltpu.VMEM((_GT,), jnp.float32),
        pltpu.SemaphoreType.DMA,
    ],
)
def _partial_sums(pred_hbm, batch_hbm, out_hbm, pbuf, bbuf, table, sem):
    wid = _wid()
    iota = lax.iota(jnp.int32, 16)
    zeros16 = jnp.zeros((16,), jnp.float32)
    nblk = _nblk(wid)

    def zero_body(g, carry):
        table[pl.ds(g * 16, 16)] = zeros16
        return carry

    lax.fori_loop(0, _G, zero_body, 0)

    def compute(k):
        off = k & 1
        pb = off * _BE
        bb = off * _BRP

        def masked_tile(tt, lo, hi, accs):
            base = pb + tt * 512
            out = list(accs)
            for v in range(8):
                rows = tt * _TILE + v * 16 + iota
                m = (rows >= lo) & (rows < hi)
                for j in range(_T):
                    out[j] = out[j] + jnp.where(
                        m, pbuf[pl.ds(base + j * _TILE + v * 16, 16)], 0.0)
            return tuple(out)

        def full_tile(tt, accs):
            base = pb + tt * 512
            out = list(accs)
            for j in range(_T):
                for v in range(8):
                    out[j] = out[j] + pbuf[
                        pl.ds(base + j * _TILE + v * 16, 16)]
            return tuple(out)

        g0 = _sload(bbuf, bb)
        g1 = _sload(bbuf, bb + _BR - 1)

        def run_body(r, p):
            g = g0 + r
            q = _lower_bound(bbuf, bb, g + 1, p)
            t0, t1, hi1, lo2 = _run_bounds(p, q)
            accs = (zeros16, zeros16, zeros16, zeros16)
            accs = masked_tile(t0, p, hi1, accs)
            accs = lax.fori_loop(t0 + 1, t1, full_tile, accs)
            accs = masked_tile(t1, lo2, q, accs)
            s0, s1, s2, s3 = (jnp.sum(a) for a in accs)
            cnt = (q - p).astype(jnp.float32)
            upd = jnp.where(
                iota == 0, s0,
                jnp.where(iota == 1, s1,
                          jnp.where(iota == 2, s2,
                                    jnp.where(iota == 3, s3,
                                              jnp.where(iota == 4, cnt,
                                                        0.0)))))
            table[pl.ds(g * 16, 16)] = table[pl.ds(g * 16, 16)] + upd
            return q

        lax.fori_loop(0, g1 - g0 + 1, run_body, jnp.int32(0))

    _issue_in(pred_hbm, batch_hbm, pbuf, bbuf, sem, wid, jnp.int32(0))
    _issue_in(pred_hbm, batch_hbm, pbuf, bbuf, sem, wid, jnp.int32(1))

    def block_body(k, carry):
        _wait_in(pred_hbm, batch_hbm, pbuf, bbuf, sem)
        compute(k)
        _issue_in(pred_hbm, batch_hbm, pbuf, bbuf, sem, wid, k + 2)
        return carry

    lax.fori_loop(0, nblk - 2, block_body, 0)
    for tail in (nblk - 2, nblk - 1):
        _wait_in(pred_hbm, batch_hbm, pbuf, bbuf, sem)
        compute(tail)
    pltpu.sync_copy(table, out_hbm.at[pl.ds(wid * _GT, _GT)])


@functools.partial(
    pl.kernel,
    out_type=jax.ShapeDtypeStruct((_GT,), jnp.float32),
    mesh=_mesh,
    compiler_params=_params,
    scratch_types=[
        pltpu.VMEM((_NW * _GPW * 16,), jnp.float32),
        pltpu.VMEM((_GPW * 16,), jnp.float32),
        pltpu.VMEM((32,), jnp.float32),
        pltpu.VMEM((_GPW * 16,), jnp.float32),
        pltpu.SemaphoreType.DMA,
    ],
)
def _correction(part_hbm, st_hbm, ms_hbm, corr_hbm, part_v, st_v, ms_v,
                out_v, sem):
    wid = _wid()
    iota = lax.iota(jnp.int32, 16)
    i4 = iota % 4
    seg = _GPW * 16

    handles = []
    for w2 in range(_NW):
        handles.append(pltpu.async_copy(
            part_hbm.at[pl.ds(w2 * _GT + wid * seg, seg)],
            part_v.at[pl.ds(w2 * seg, seg)], sem))
    handles.append(pltpu.async_copy(st_hbm.at[pl.ds(wid * seg, seg)], st_v,
                                    sem))
    handles.append(pltpu.async_copy(ms_hbm, ms_v, sem))
    for h in handles:
        h.wait()

    def seg_body(j, carry):
        def add_w(w2, acc):
            return acc + part_v[pl.ds(w2 * seg + j * 16, 16)]

        acc = lax.fori_loop(0, _NW, add_w, jnp.zeros((16,), jnp.float32))
        s0 = jnp.sum(jnp.where(iota == 0, acc, 0.0))
        s1 = jnp.sum(jnp.where(iota == 1, acc, 0.0))
        s2 = jnp.sum(jnp.where(iota == 2, acc, 0.0))
        s3 = jnp.sum(jnp.where(iota == 3, acc, 0.0))
        cnt = jnp.sum(jnp.where(iota == 4, acc, 0.0))
        psum = jnp.where(i4 == 0, s0,
                         jnp.where(i4 == 1, s1,
                                   jnp.where(i4 == 2, s2, s3)))
        st = st_v[pl.ds(j * 16, 16)]
        meanv = ms_v[pl.ds(0, 16)]
        stdv = ms_v[pl.ds(16, 16)]
        corr = ((st - cnt * meanv) / stdv - psum) / cnt
        out_v[pl.ds(j * 16, 16)] = corr
        return carry

    lax.fori_loop(0, _GPW, seg_body, 0)
    pltpu.sync_copy(out_v, corr_hbm.at[pl.ds(wid * seg, seg)])


@functools.partial(
    pl.kernel,
    out_type=jax.ShapeDtypeStruct((_N * _T,), jnp.float32),
    mesh=_mesh,
    compiler_params=_params,
    scratch_types=[
        pltpu.VMEM((2 * _BE,), jnp.float32),
        pltpu.VMEM((2 * _BRP,), jnp.int32),
        pltpu.VMEM((_GT,), jnp.float32),
        pltpu.SemaphoreType.DMA,
    ],
)
def _apply(pred_hbm, batch_hbm, corr_hbm, out_hbm, pbuf, bbuf, corr_v,
           sem_in):
    wid = _wid()
    iota = lax.iota(jnp.int32, 16)
    nblk = _nblk(wid)
    pltpu.sync_copy(corr_hbm, corr_v)

    def compute(k):
        off = k & 1
        pb = off * _BE
        bb = off * _BRP

        def masked_tile(tt, lo, hi, cj):
            for v in range(8):
                rows = tt * _TILE + v * 16 + iota
                m = (rows >= lo) & (rows < hi)
                for j in range(_T):
                    o = pb + tt * 512 + j * _TILE + v * 16
                    pbuf[pl.ds(o, 16)] = (
                        pbuf[pl.ds(o, 16)] + jnp.where(m, cj[j], 0.0))

        g0 = _sload(bbuf, bb)
        g1 = _sload(bbuf, bb + _BR - 1)

        def run_body(r, p):
            g = g0 + r
            q = _lower_bound(bbuf, bb, g + 1, p)
            t0, t1, hi1, lo2 = _run_bounds(p, q)
            cvec = corr_v[pl.ds(g * 16, 16)]
            cj = tuple(jnp.full((16,), cvec[j], jnp.float32)
                       for j in range(_T))

            def full_tile(tt, carry2):
                for j in range(_T):
                    for v in range(8):
                        o = pb + tt * 512 + j * _TILE + v * 16
                        pbuf[pl.ds(o, 16)] = pbuf[pl.ds(o, 16)] + cj[j]
                return carry2

            masked_tile(t0, p, hi1, cj)
            lax.fori_loop(t0 + 1, t1, full_tile, 0)
            masked_tile(t1, lo2, q, cj)
            return q

        lax.fori_loop(0, g1 - g0 + 1, run_body, jnp.int32(0))

    def flush(k):
        ebase = (wid + k * _NW) * _BE
        pltpu.sync_copy(pbuf.at[pl.ds((k & 1) * _BE, _BE)],
                        out_hbm.at[pl.ds(ebase, _BE)])

    _issue_in(pred_hbm, batch_hbm, pbuf, bbuf, sem_in, wid, jnp.int32(0))
    _issue_in(pred_hbm, batch_hbm, pbuf, bbuf, sem_in, wid, jnp.int32(1))

    def block_body(k, carry):
        _wait_in(pred_hbm, batch_hbm, pbuf, bbuf, sem_in)
        compute(k)
        flush(k)
        _issue_in(pred_hbm, batch_hbm, pbuf, bbuf, sem_in, wid, k + 2)
        return carry

    lax.fori_loop(0, nblk - 2, block_body, 0)
    for tail in (nblk - 2, nblk - 1):
        _wait_in(pred_hbm, batch_hbm, pbuf, bbuf, sem_in)
        compute(tail)
        flush(tail)


def kernel(pred, batch, sum_target, mean, std):
    pred_flat = pred.reshape(_NTILES, _TILE, _T).transpose(0, 2, 1).reshape(-1)
    st16 = jnp.tile(sum_target, (1, 4)).reshape(-1)
    ms = jnp.concatenate([jnp.tile(mean, 4), jnp.tile(std, 4)])
    part = _partial_sums(pred_flat, batch)
    corr = _correction(part, st16, ms)
    out = _apply(pred_flat, batch, corr)
    return out.reshape(_NTILES, _T, _TILE).transpose(0, 2, 1).reshape(_N, _T)


# speedup vs baseline: 2.0113x; 1.0002x over previous
import functools

import jax
import jax.numpy as jnp
from jax import lax
from jax.experimental import pallas as pl
from jax.experimental.pallas import tpu as pltpu
from jax.experimental.pallas import tpu_sc as plsc

_N = 6_400_000
_T = 4
_G = 512
_NC = 2
_NS = 16
_NW = _NC * _NS
_TILE = 128
_NTILES = _N // _TILE
_TPB = 40
_BR = _TPB * _TILE
_BRP = _BR + 16
_BE = _BR * _T
_NBLK = _NTILES // _TPB
_BLK_Q, _BLK_R = divmod(_NBLK, _NW)
_GT = _G * 16
_GPW = _G // _NW
_BS_ITERS = 13

_mesh = plsc.VectorSubcoreMesh(core_axis_name="c", subcore_axis_name="s")
_params = pltpu.CompilerParams(needs_layout_passes=False)


def _wid():
    return lax.axis_index("s") * _NC + lax.axis_index("c")


def _nblk(wid):
    return _BLK_Q + (wid < _BLK_R).astype(jnp.int32)


def _sload(ref, i):
    return ref[pl.ds(i, 16)][0]


def _lower_bound(bbuf, bb, x, lo0):
    def body(_, c):
        lo, hi = c
        mid = (lo + hi) >> 1
        act = lo < hi
        lt = _sload(bbuf, bb + mid) < x
        lo = jnp.where(act & lt, mid + 1, lo)
        hi = jnp.where(act & (~lt), mid, hi)
        return lo, hi

    lo, _ = lax.fori_loop(0, _BS_ITERS, body, (lo0, jnp.int32(_BR)))
    return lo


def _run_bounds(p, q):
    t0 = p >> 7
    t1 = jnp.maximum(q - 1, p) >> 7
    hi1 = jnp.minimum(q, (t0 + 1) * _TILE)
    lo2 = jnp.where(t1 > t0, t1 * _TILE, q)
    return t0, t1, hi1, lo2


def _issue_in(pred_hbm, batch_hbm, pbuf, bbuf, sem, wid, k, nbuf=1):
    blk = wid + k * _NW
    off = k & nbuf
    rbase = blk * _BR
    ebase = rbase * _T
    pltpu.async_copy(pred_hbm.at[pl.ds(ebase, _BE)],
                     pbuf.at[pl.ds(off * _BE, _BE)], sem)
    pltpu.async_copy(batch_hbm.at[pl.ds(rbase, _BR)],
                     bbuf.at[pl.ds(off * _BRP, _BR)], sem)


def _wait_in(pred_hbm, batch_hbm, pbuf, bbuf, sem):
    pltpu.make_async_copy(pred_hbm.at[pl.ds(0, _BE)],
                          pbuf.at[pl.ds(0, _BE)], sem).wait()
    pltpu.make_async_copy(batch_hbm.at[pl.ds(0, _BR)],
                          bbuf.at[pl.ds(0, _BR)], sem).wait()


@functools.partial(
    pl.kernel,
    out_type=jax.ShapeDtypeStruct((_NW * _GT,), jnp.float32),
    mesh=_mesh,
    compiler_params=_params,
    scratch_types=[
        pltpu.VMEM((2 * _BE,), jnp.float32),
        pltpu.VMEM((2 * _BRP,), jnp.int32),
        pltpu.VMEM((_GT,), jnp.float32),
        pltpu.SemaphoreType.DMA,
    ],
)
def _partial_sums(pred_hbm, batch_hbm, out_hbm, pbuf, bbuf, table, sem):
    wid = _wid()
    iota = lax.iota(jnp.int32, 16)
    zeros16 = jnp.zeros((16,), jnp.float32)
    nblk = _nblk(wid)

    def zero_body(g, carry):
        table[pl.ds(g * 16, 16)] = zeros16
        return carry

    lax.fori_loop(0, _G, zero_body, 0)

    def compute(k):
        off = k & 1
        pb = off * _BE
        bb = off * _BRP

        def masked_tile(tt, lo, hi, accs):
            base = pb + tt * 512
            out = list(accs)
            for v in range(8):
                rows = tt * _TILE + v * 16 + iota
                m = (rows >= lo) & (rows < hi)
                for j in range(_T):
                    out[j] = out[j] + jnp.where(
                        m, pbuf[pl.ds(base + j * _TILE + v * 16, 16)], 0.0)
            return tuple(out)

        def full_tile(tt, accs):
            base = pb + tt * 512
            out = list(accs)
            for j in range(_T):
                for v in range(8):
                    out[j] = out[j] + pbuf[
                        pl.ds(base + j * _TILE + v * 16, 16)]
            return tuple(out)

        g0 = _sload(bbuf, bb)
        g1 = _sload(bbuf, bb + _BR - 1)

        def run_body(r, p):
            g = g0 + r
            q = _lower_bound(bbuf, bb, g + 1, p)
            t0, t1, hi1, lo2 = _run_bounds(p, q)
            accs = (zeros16, zeros16, zeros16, zeros16)
            accs = masked_tile(t0, p, hi1, accs)
            accs = lax.fori_loop(t0 + 1, t1, full_tile, accs)
            accs = masked_tile(t1, lo2, q, accs)
            s0, s1, s2, s3 = (jnp.sum(a) for a in accs)
            cnt = (q - p).astype(jnp.float32)
            upd = jnp.where(
                iota == 0, s0,
                jnp.where(iota == 1, s1,
                          jnp.where(iota == 2, s2,
                                    jnp.where(iota == 3, s3,
                                              jnp.where(iota == 4, cnt,
                                                        0.0)))))
            table[pl.ds(g * 16, 16)] = table[pl.ds(g * 16, 16)] + upd
            return q

        lax.fori_loop(0, g1 - g0 + 1, run_body, jnp.int32(0))

    _issue_in(pred_hbm, batch_hbm, pbuf, bbuf, sem, wid, jnp.int32(0))
    _issue_in(pred_hbm, batch_hbm, pbuf, bbuf, sem, wid, jnp.int32(1))

    def block_body(k, carry):
        _wait_in(pred_hbm, batch_hbm, pbuf, bbuf, sem)
        compute(k)
        _issue_in(pred_hbm, batch_hbm, pbuf, bbuf, sem, wid, k + 2)
        return carry

    lax.fori_loop(0, nblk - 2, block_body, 0)
    for tail in (nblk - 2, nblk - 1):
        _wait_in(pred_hbm, batch_hbm, pbuf, bbuf, sem)
        compute(tail)
    pltpu.sync_copy(table, out_hbm.at[pl.ds(wid * _GT, _GT)])


@functools.partial(
    pl.kernel,
    out_type=jax.ShapeDtypeStruct((_GT,), jnp.float32),
    mesh=_mesh,
    compiler_params=_params,
    scratch_types=[
        pltpu.VMEM((_NW * _GPW * 16,), jnp.float32),
        pltpu.VMEM((_GPW * 16,), jnp.float32),
        pltpu.VMEM((32,), jnp.float32),
        pltpu.VMEM((_GPW * 16,), jnp.float32),
        pltpu.SemaphoreType.DMA,
    ],
)
def _correction(part_hbm, st_hbm, ms_hbm, corr_hbm, part_v, st_v, ms_v,
                out_v, sem):
    wid = _wid()
    iota = lax.iota(jnp.int32, 16)
    i4 = iota % 4
    seg = _GPW * 16

    handles = []
    for w2 in range(_NW):
        handles.append(pltpu.async_copy(
            part_hbm.at[pl.ds(w2 * _GT + wid * seg, seg)],
            part_v.at[pl.ds(w2 * seg, seg)], sem))
    handles.append(pltpu.async_copy(st_hbm.at[pl.ds(wid * seg, seg)], st_v,
                                    sem))
    handles.append(pltpu.async_copy(ms_hbm, ms_v, sem))
    for h in handles:
        h.wait()

    def seg_body(j, carry):
        def add_w(w2, acc):
            return acc + part_v[pl.ds(w2 * seg + j * 16, 16)]

        acc = lax.fori_loop(0, _NW, add_w, jnp.zeros((16,), jnp.float32))
        s0 = jnp.sum(jnp.where(iota == 0, acc, 0.0))
        s1 = jnp.sum(jnp.where(iota == 1, acc, 0.0))
        s2 = jnp.sum(jnp.where(iota == 2, acc, 0.0))
        s3 = jnp.sum(jnp.where(iota == 3, acc, 0.0))
        cnt = jnp.sum(jnp.where(iota == 4, acc, 0.0))
        psum = jnp.where(i4 == 0, s0,
                         jnp.where(i4 == 1, s1,
                                   jnp.where(i4 == 2, s2, s3)))
        st = st_v[pl.ds(j * 16, 16)]
        meanv = ms_v[pl.ds(0, 16)]
        stdv = ms_v[pl.ds(16, 16)]
        corr = ((st - cnt * meanv) / stdv - psum) / cnt
        out_v[pl.ds(j * 16, 16)] = corr
        return carry

    lax.fori_loop(0, _GPW, seg_body, 0)
    pltpu.sync_copy(out_v, corr_hbm.at[pl.ds(wid * seg, seg)])


@functools.partial(
    pl.kernel,
    out_type=jax.ShapeDtypeStruct((_N * _T,), jnp.float32),
    mesh=_mesh,
    compiler_params=_params,
    scratch_types=[
        pltpu.VMEM((4 * _BE,), jnp.float32),
        pltpu.VMEM((4 * _BRP,), jnp.int32),
        pltpu.VMEM((_GT,), jnp.float32),
        pltpu.SemaphoreType.DMA,
        pltpu.SemaphoreType.DMA,
    ],
)
def _apply(pred_hbm, batch_hbm, corr_hbm, out_hbm, pbuf, bbuf, corr_v,
           sem_in, sem_out):
    wid = _wid()
    iota = lax.iota(jnp.int32, 16)
    nblk = _nblk(wid)
    pltpu.sync_copy(corr_hbm, corr_v)

    def compute(k):
        off = k & 3
        pb = off * _BE
        bb = off * _BRP

        def masked_tile(tt, lo, hi, cj):
            for v in range(8):
                rows = tt * _TILE + v * 16 + iota
                m = (rows >= lo) & (rows < hi)
                for j in range(_T):
                    o = pb + tt * 512 + j * _TILE + v * 16
                    pbuf[pl.ds(o, 16)] = (
                        pbuf[pl.ds(o, 16)] + jnp.where(m, cj[j], 0.0))

        g0 = _sload(bbuf, bb)
        g1 = _sload(bbuf, bb + _BR - 1)

        def run_body(r, p):
            g = g0 + r
            q = _lower_bound(bbuf, bb, g + 1, p)
            t0, t1, hi1, lo2 = _run_bounds(p, q)
            cvec = corr_v[pl.ds(g * 16, 16)]
            cj = tuple(jnp.full((16,), cvec[j], jnp.float32)
                       for j in range(_T))

            def full_tile(tt, carry2):
                for j in range(_T):
                    for v in range(8):
                        o = pb + tt * 512 + j * _TILE + v * 16
                        pbuf[pl.ds(o, 16)] = pbuf[pl.ds(o, 16)] + cj[j]
                return carry2

            masked_tile(t0, p, hi1, cj)
            lax.fori_loop(t0 + 1, t1, full_tile, 0)
            masked_tile(t1, lo2, q, cj)
            return q

        lax.fori_loop(0, g1 - g0 + 1, run_body, jnp.int32(0))

    def issue_out(k):
        ebase = (wid + k * _NW) * _BE
        pltpu.async_copy(pbuf.at[pl.ds((k & 3) * _BE, _BE)],
                         out_hbm.at[pl.ds(ebase, _BE)], sem_out)

    def wait_out():
        pltpu.make_async_copy(pbuf.at[pl.ds(0, _BE)],
                              out_hbm.at[pl.ds(0, _BE)], sem_out).wait()

    _issue_in(pred_hbm, batch_hbm, pbuf, bbuf, sem_in, wid, jnp.int32(0), 3)
    _issue_in(pred_hbm, batch_hbm, pbuf, bbuf, sem_in, wid, jnp.int32(1), 3)

    for k0 in (0, 1):
        _wait_in(pred_hbm, batch_hbm, pbuf, bbuf, sem_in)
        compute(jnp.int32(k0))
        issue_out(jnp.int32(k0))
        _issue_in(pred_hbm, batch_hbm, pbuf, bbuf, sem_in, wid,
                  jnp.int32(k0 + 2), 3)

    def block_body(k, carry):
        _wait_in(pred_hbm, batch_hbm, pbuf, bbuf, sem_in)
        compute(k)
        issue_out(k)
        wait_out()
        _issue_in(pred_hbm, batch_hbm, pbuf, bbuf, sem_in, wid, k + 2, 3)
        return carry

    lax.fori_loop(2, nblk - 2, block_body, 0)
    for tail in (nblk - 2, nblk - 1):
        _wait_in(pred_hbm, batch_hbm, pbuf, bbuf, sem_in)
        compute(tail)
        issue_out(tail)
    for _ in range(4):
        wait_out()


def kernel(pred, batch, sum_target, mean, std):
    pred_flat = pred.reshape(_NTILES, _TILE, _T).transpose(0, 2, 1).reshape(-1)
    st16 = jnp.tile(sum_target, (1, 4)).reshape(-1)
    ms = jnp.concatenate([jnp.tile(mean, 4), jnp.tile(std, 4)])
    part = _partial_sums(pred_flat, batch)
    corr = _correction(part, st16, ms)
    out = _apply(pred_flat, batch, corr)
    return out.reshape(_NTILES, _T, _TILE).transpose(0, 2, 1).reshape(_N, _T)


# speedup vs baseline: 2.0786x; 1.0335x over previous
import functools

import jax
import jax.numpy as jnp
from jax import lax
from jax.experimental import pallas as pl
from jax.experimental.pallas import tpu as pltpu
from jax.experimental.pallas import tpu_sc as plsc

_N = 6_400_000
_T = 4
_G = 512
_NC = 2
_NS = 16
_NW = _NC * _NS
_TILE = 128
_NTILES = _N // _TILE
_TPB = 40
_BR = _TPB * _TILE
_BRP = _BR + 16
_BE = _BR * _T
_NBLK = _NTILES // _TPB
_BLK_Q, _BLK_R = divmod(_NBLK, _NW)
_GT = _G * 16
_GPW = _G // _NW
_BS_ITERS = 13

_mesh = plsc.VectorSubcoreMesh(core_axis_name="c", subcore_axis_name="s")
_params = pltpu.CompilerParams(needs_layout_passes=False)


def _wid():
    return lax.axis_index("s") * _NC + lax.axis_index("c")


def _nblk(wid):
    return _BLK_Q + (wid < _BLK_R).astype(jnp.int32)


def _sload(ref, i):
    return ref[pl.ds(i, 16)][0]


def _lower_bound(bbuf, bb, x, lo0):
    def body(_, c):
        lo, hi = c
        mid = (lo + hi) >> 1
        act = lo < hi
        lt = _sload(bbuf, bb + mid) < x
        lo = jnp.where(act & lt, mid + 1, lo)
        hi = jnp.where(act & (~lt), mid, hi)
        return lo, hi

    lo, _ = lax.fori_loop(0, _BS_ITERS, body, (lo0, jnp.int32(_BR)))
    return lo


def _run_bounds(p, q):
    t0 = p >> 7
    t1 = jnp.maximum(q - 1, p) >> 7
    hi1 = jnp.minimum(q, (t0 + 1) * _TILE)
    lo2 = jnp.where(t1 > t0, t1 * _TILE, q)
    return t0, t1, hi1, lo2


def _issue_in(pred_hbm, batch_hbm, pbuf, bbuf, sem, wid, k, nbuf=1):
    blk = wid + k * _NW
    off = k & nbuf
    rbase = blk * _BR
    ebase = rbase * _T
    pltpu.async_copy(pred_hbm.at[pl.ds(ebase, _BE)],
                     pbuf.at[pl.ds(off * _BE, _BE)], sem)
    pltpu.async_copy(batch_hbm.at[pl.ds(rbase, _BR)],
                     bbuf.at[pl.ds(off * _BRP, _BR)], sem)


def _wait_in(pred_hbm, batch_hbm, pbuf, bbuf, sem):
    pltpu.make_async_copy(pred_hbm.at[pl.ds(0, _BE)],
                          pbuf.at[pl.ds(0, _BE)], sem).wait()
    pltpu.make_async_copy(batch_hbm.at[pl.ds(0, _BR)],
                          bbuf.at[pl.ds(0, _BR)], sem).wait()


@functools.partial(
    pl.kernel,
    out_type=jax.ShapeDtypeStruct((_NW * _GT,), jnp.float32),
    mesh=_mesh,
    compiler_params=_params,
    scratch_types=[
        pltpu.VMEM((2 * _BE,), jnp.float32),
        pltpu.VMEM((2 * _BRP,), jnp.int32),
        pltpu.VMEM((_GT,), jnp.float32),
        pltpu.SemaphoreType.DMA,
    ],
)
def _partial_sums(pred_hbm, batch_hbm, out_hbm, pbuf, bbuf, table, sem):
    wid = _wid()
    iota = lax.iota(jnp.int32, 16)
    zeros16 = jnp.zeros((16,), jnp.float32)
    nblk = _nblk(wid)

    def zero_body(g, carry):
        table[pl.ds(g * 16, 16)] = zeros16
        return carry

    lax.fori_loop(0, _G, zero_body, 0)

    def compute(k):
        off = k & 1
        pb = off * _BE
        bb = off * _BRP

        def masked_tile(tt, lo, hi, accs):
            base = pb + tt * 512
            out = list(accs)
            for v in range(8):
                rows = tt * _TILE + v * 16 + iota
                m = (rows >= lo) & (rows < hi)
                for j in range(_T):
                    out[j] = out[j] + jnp.where(
                        m, pbuf[pl.ds(base + j * _TILE + v * 16, 16)], 0.0)
            return tuple(out)

        def full_tile(tt, accs):
            base = pb + tt * 512
            out = list(accs)
            for j in range(_T):
                for v in range(8):
                    out[j] = out[j] + pbuf[
                        pl.ds(base + j * _TILE + v * 16, 16)]
            return tuple(out)

        g0 = _sload(bbuf, bb)
        g1 = _sload(bbuf, bb + _BR - 1)

        def run_body(r, p):
            g = g0 + r
            q = _lower_bound(bbuf, bb, g + 1, p)
            t0, t1, hi1, lo2 = _run_bounds(p, q)
            accs = (zeros16, zeros16, zeros16, zeros16)
            accs = masked_tile(t0, p, hi1, accs)
            accs = lax.fori_loop(t0 + 1, t1, full_tile, accs)
            accs = masked_tile(t1, lo2, q, accs)
            s0, s1, s2, s3 = (jnp.sum(a) for a in accs)
            cnt = (q - p).astype(jnp.float32)
            upd = jnp.where(
                iota == 0, s0,
                jnp.where(iota == 1, s1,
                          jnp.where(iota == 2, s2,
                                    jnp.where(iota == 3, s3,
                                              jnp.where(iota == 4, cnt,
                                                        0.0)))))
            table[pl.ds(g * 16, 16)] = table[pl.ds(g * 16, 16)] + upd
            return q

        lax.fori_loop(0, g1 - g0 + 1, run_body, jnp.int32(0))

    _issue_in(pred_hbm, batch_hbm, pbuf, bbuf, sem, wid, jnp.int32(0))
    _issue_in(pred_hbm, batch_hbm, pbuf, bbuf, sem, wid, jnp.int32(1))

    def block_body(k, carry):
        _wait_in(pred_hbm, batch_hbm, pbuf, bbuf, sem)
        compute(k)
        _issue_in(pred_hbm, batch_hbm, pbuf, bbuf, sem, wid, k + 2)
        return carry

    lax.fori_loop(0, nblk - 2, block_body, 0)
    for tail in (nblk - 2, nblk - 1):
        _wait_in(pred_hbm, batch_hbm, pbuf, bbuf, sem)
        compute(tail)
    pltpu.sync_copy(table, out_hbm.at[pl.ds(wid * _GT, _GT)])


@functools.partial(
    pl.kernel,
    out_type=(jax.ShapeDtypeStruct((_GT,), jnp.float32),
              jax.ShapeDtypeStruct((_G,), jnp.float32)),
    mesh=_mesh,
    compiler_params=_params,
    scratch_types=[
        pltpu.VMEM((_NW * _GPW * 16,), jnp.float32),
        pltpu.VMEM((_GPW * 16,), jnp.float32),
        pltpu.VMEM((32,), jnp.float32),
        pltpu.VMEM((_GPW * 16,), jnp.float32),
        pltpu.VMEM((16,), jnp.float32),
        pltpu.SemaphoreType.DMA,
    ],
)
def _correction(part_hbm, st_hbm, ms_hbm, corr_hbm, counts_hbm, part_v, st_v,
                ms_v, out_v, cnt_v, sem):
    wid = _wid()
    iota = lax.iota(jnp.int32, 16)
    i4 = iota % 4
    seg = _GPW * 16

    handles = []
    for w2 in range(_NW):
        handles.append(pltpu.async_copy(
            part_hbm.at[pl.ds(w2 * _GT + wid * seg, seg)],
            part_v.at[pl.ds(w2 * seg, seg)], sem))
    handles.append(pltpu.async_copy(st_hbm.at[pl.ds(wid * seg, seg)], st_v,
                                    sem))
    handles.append(pltpu.async_copy(ms_hbm, ms_v, sem))
    for h in handles:
        h.wait()

    def seg_body(j, cnts):
        def add_w(w2, acc):
            return acc + part_v[pl.ds(w2 * seg + j * 16, 16)]

        acc = lax.fori_loop(0, _NW, add_w, jnp.zeros((16,), jnp.float32))
        s0 = jnp.sum(jnp.where(iota == 0, acc, 0.0))
        s1 = jnp.sum(jnp.where(iota == 1, acc, 0.0))
        s2 = jnp.sum(jnp.where(iota == 2, acc, 0.0))
        s3 = jnp.sum(jnp.where(iota == 3, acc, 0.0))
        cnt = jnp.sum(jnp.where(iota == 4, acc, 0.0))
        psum = jnp.where(i4 == 0, s0,
                         jnp.where(i4 == 1, s1,
                                   jnp.where(i4 == 2, s2, s3)))
        st = st_v[pl.ds(j * 16, 16)]
        meanv = ms_v[pl.ds(0, 16)]
        stdv = ms_v[pl.ds(16, 16)]
        corr = ((st - cnt * meanv) / stdv - psum) / cnt
        out_v[pl.ds(j * 16, 16)] = corr
        return jnp.where(iota == j, cnt, cnts)

    cnts = lax.fori_loop(0, _GPW, seg_body, jnp.zeros((16,), jnp.float32))
    cnt_v[pl.ds(0, 16)] = cnts
    pltpu.sync_copy(out_v, corr_hbm.at[pl.ds(wid * seg, seg)])
    pltpu.sync_copy(cnt_v, counts_hbm.at[pl.ds(wid * _GPW, _GPW)])


@functools.partial(
    pl.kernel,
    out_type=jax.ShapeDtypeStruct((_N * _T,), jnp.float32),
    mesh=_mesh,
    compiler_params=_params,
    scratch_types=[
        pltpu.VMEM((4 * _BE,), jnp.float32),
        pltpu.VMEM((_GT,), jnp.float32),
        pltpu.VMEM((_G,), jnp.float32),
        pltpu.VMEM((_G + 32,), jnp.int32),
        pltpu.SemaphoreType.DMA,
        pltpu.SemaphoreType.DMA,
    ],
)
def _apply(pred_hbm, corr_hbm, counts_hbm, out_hbm, pbuf, corr_v, cnt_v,
           cum_v, sem_in, sem_out):
    wid = _wid()
    iota = lax.iota(jnp.int32, 16)
    nblk = _nblk(wid)
    pltpu.sync_copy(corr_hbm, corr_v)
    pltpu.sync_copy(counts_hbm, cnt_v)

    cum_v[pl.ds(0, 16)] = jnp.zeros((16,), jnp.int32)
    carry = jnp.int32(0)
    for c in range(_G // 16):
        v = cnt_v[pl.ds(c * 16, 16)].astype(jnp.int32)
        cs = plsc.cumsum(v) + carry
        cum_v[pl.ds(c * 16 + 1, 16)] = cs
        carry = cs[15]

    def _lb_cum(x, inclusive):
        def body(_, c):
            lo, hi = c
            mid = (lo + hi) >> 1
            act = lo < hi
            val = _sload(cum_v, 1 + mid)
            go_right = (val <= x) if inclusive else (val < x)
            lo = jnp.where(act & go_right, mid + 1, lo)
            hi = jnp.where(act & (~go_right), mid, hi)
            return lo, hi

        lo, _ = lax.fori_loop(0, 9, body, (jnp.int32(0), jnp.int32(_G)))
        return lo

    def compute(k):
        off = k & 3
        pb = off * _BE

        def masked_tile(tt, lo, hi, cj):
            for v in range(8):
                rows = tt * _TILE + v * 16 + iota
                m = (rows >= lo) & (rows < hi)
                for j in range(_T):
                    o = pb + tt * 512 + j * _TILE + v * 16
                    pbuf[pl.ds(o, 16)] = (
                        pbuf[pl.ds(o, 16)] + jnp.where(m, cj[j], 0.0))

        rbase = (wid + k * _NW) * _BR
        rend = rbase + _BR
        g0 = _lb_cum(rbase, True)
        g1 = _lb_cum(rend - 1, True)

        def run_body(r, carry1):
            g = g0 + r
            p = jnp.maximum(_sload(cum_v, g) - rbase, 0)
            q = jnp.minimum(_sload(cum_v, g + 1) - rbase, _BR)
            t0, t1, hi1, lo2 = _run_bounds(p, q)
            cvec = corr_v[pl.ds(g * 16, 16)]
            cj = tuple(jnp.full((16,), cvec[j], jnp.float32)
                       for j in range(_T))

            def full_tile(tt, carry2):
                for j in range(_T):
                    for v in range(8):
                        o = pb + tt * 512 + j * _TILE + v * 16
                        pbuf[pl.ds(o, 16)] = pbuf[pl.ds(o, 16)] + cj[j]
                return carry2

            masked_tile(t0, p, hi1, cj)
            lax.fori_loop(t0 + 1, t1, full_tile, 0)
            masked_tile(t1, lo2, q, cj)
            return carry1

        lax.fori_loop(0, g1 - g0 + 1, run_body, 0)

    def issue_in(k):
        ebase = (wid + k * _NW) * _BE
        pltpu.async_copy(pred_hbm.at[pl.ds(ebase, _BE)],
                         pbuf.at[pl.ds((k & 3) * _BE, _BE)], sem_in)

    def wait_in():
        pltpu.make_async_copy(pred_hbm.at[pl.ds(0, _BE)],
                              pbuf.at[pl.ds(0, _BE)], sem_in).wait()

    def issue_out(k):
        ebase = (wid + k * _NW) * _BE
        pltpu.async_copy(pbuf.at[pl.ds((k & 3) * _BE, _BE)],
                         out_hbm.at[pl.ds(ebase, _BE)], sem_out)

    def wait_out():
        pltpu.make_async_copy(pbuf.at[pl.ds(0, _BE)],
                              out_hbm.at[pl.ds(0, _BE)], sem_out).wait()

    issue_in(jnp.int32(0))
    issue_in(jnp.int32(1))

    for k0 in (0, 1):
        wait_in()
        compute(jnp.int32(k0))
        issue_out(jnp.int32(k0))
        issue_in(jnp.int32(k0 + 2))

    def block_body(k, carry):
        wait_in()
        compute(k)
        issue_out(k)
        wait_out()
        issue_in(k + 2)
        return carry

    lax.fori_loop(2, nblk - 2, block_body, 0)
    for tail in (nblk - 2, nblk - 1):
        wait_in()
        compute(tail)
        issue_out(tail)
    for _ in range(4):
        wait_out()


def kernel(pred, batch, sum_target, mean, std):
    pred_flat = pred.reshape(_NTILES, _TILE, _T).transpose(0, 2, 1).reshape(-1)
    st16 = jnp.tile(sum_target, (1, 4)).reshape(-1)
    ms = jnp.concatenate([jnp.tile(mean, 4), jnp.tile(std, 4)])
    part = _partial_sums(pred_flat, batch)
    corr, counts = _correction(part, st16, ms)
    out = _apply(pred_flat, corr, counts)
    return out.reshape(_NTILES, _T, _TILE).transpose(0, 2, 1).reshape(_N, _T)
